# baseline (speedup 1.0000x reference)
import functools

import jax
import jax.numpy as jnp
from jax import lax
from jax.experimental import pallas as pl
from jax.experimental.pallas import tpu as pltpu
from jax.experimental.pallas import tpu_sc as plsc

B = 8
N = 1250
NP = 1280
E = 20000
EP = 20480
IN_C = 256
HID = 512
OUT_C = 256

NC = 2
NS = 16
CHUNKS = 10
CELLS = NP * NP
SLICE = CELLS // NS
PAD_CELL = CELLS


def _leaky(x):
    return jnp.where(x > 0, x, 0.01 * x)


def _aggr_dot(a, x):
    bf = jnp.bfloat16
    ab = a.astype(bf)
    xh = x.astype(bf)
    r1 = x - xh.astype(jnp.float32)
    xm = r1.astype(bf)
    xl = (r1 - xm.astype(jnp.float32)).astype(bf)
    acc = jnp.dot(ab, xh, preferred_element_type=jnp.float32)
    acc += jnp.dot(ab, xm, preferred_element_type=jnp.float32)
    acc += jnp.dot(ab, xl, preferred_element_type=jnp.float32)
    return acc


def _build_adjacency(idx, zeros_src):
    mesh = plsc.VectorSubcoreMesh(
        core_axis_name="c", subcore_axis_name="s",
        num_cores=NC, num_subcores=NS)

    @functools.partial(
        pl.kernel,
        mesh=mesh,
        out_type=jax.ShapeDtypeStruct((B, NP, NP), jnp.float32),
        scratch_types=[
            pltpu.VMEM((CHUNKS, 128), jnp.int32),
            pltpu.VMEM((128,), jnp.float32),
            pltpu.VMEM((16, NP), jnp.float32),
            pltpu.VMEM_SHARED((CELLS + 16,), jnp.float32),
        ],
    )
    def build(idx_hbm, zeros_hbm, a_hbm, idx_v, ones_v, row_v, a_sh):
        cid = lax.axis_index("c")
        sid = lax.axis_index("s")
        rows = NP // NS
        for i in range(8):
            ones_v[pl.ds(16 * i, 16)] = jnp.ones((16,), jnp.float32)
        pltpu.sync_copy(zeros_hbm, a_sh.at[pl.ds(sid * SLICE, SLICE)])
        plsc.subcore_barrier()
        for bi in range(B // NC):
            b = cid * (B // NC) + bi
            pltpu.sync_copy(idx_hbm.at[b * NS + sid], idx_v)
            for j in range(CHUNKS):
                pltpu.sync_copy(ones_v, a_sh.at[idx_v.at[j]], add=True)
            plsc.subcore_barrier()
            for g in range(rows // 16):
                def _row(r, _):
                    pltpu.sync_copy(
                        a_sh.at[pl.ds((sid * rows + g * 16 + r) * NP, NP)],
                        row_v.at[r])
                    return _
                lax.fori_loop(0, 16, _row, 0)
                pltpu.sync_copy(
                    row_v, a_hbm.at[b].at[pl.ds(sid * rows + g * 16, 16)])
            if bi != B // NC - 1:
                pltpu.sync_copy(zeros_hbm, a_sh.at[pl.ds(sid * SLICE, SLICE)])
                plsc.subcore_barrier()

    return build(idx, zeros_src)


RB = 640


def _conv_kernel_first(a_ref, xf_ref, w1_ref, b1_ref, w2_ref, b2_ref,
                       eps_ref, out_ref):
    xf = xf_ref[0]
    aggr = _aggr_dot(a_ref[0][:N, :N], xf)
    h = aggr + (1.0 + eps_ref[0, 0]) * xf
    t = jnp.dot(h, w1_ref[...], preferred_element_type=jnp.float32)
    t = _leaky(t + b1_ref[...])
    o = jnp.dot(t, w2_ref[...], preferred_element_type=jnp.float32)
    o = _leaky(o + b2_ref[...])
    out_ref[0] = jnp.concatenate(
        [o, jnp.zeros((NP - N, HID), jnp.float32)], axis=0)


def _conv_layer_first(a, x, w1, b1, w2, b2, eps):
    d = x.shape[-1]
    return pl.pallas_call(
        _conv_kernel_first,
        grid=(B,),
        in_specs=[
            pl.BlockSpec((1, NP, NP), lambda b: (b, 0, 0)),
            pl.BlockSpec((1, N, d), lambda b: (b, 0, 0)),
            pl.BlockSpec((d, HID), lambda b: (0, 0)),
            pl.BlockSpec((1, HID), lambda b: (0, 0)),
            pl.BlockSpec((HID, HID), lambda b: (0, 0)),
            pl.BlockSpec((1, HID), lambda b: (0, 0)),
            pl.BlockSpec((1, 128), lambda b: (0, 0)),
        ],
        out_specs=pl.BlockSpec((1, NP, HID), lambda b: (b, 0, 0)),
        out_shape=jax.ShapeDtypeStruct((B, NP, HID), jnp.float32),
    )(a, x, w1, b1.reshape(1, HID), w2, b2.reshape(1, HID),
      jnp.full((1, 128), eps, jnp.float32))


def _conv_kernel(a_ref, xf_ref, w1_ref, b1_ref, w2_ref, b2_ref,
                 eps_ref, out_ref):
    r = pl.program_id(1)
    xf = xf_ref[0]
    aggr = _aggr_dot(a_ref[0], xf)
    xb = xf_ref[0, pl.ds(r * RB, RB), :]
    h = aggr + (1.0 + eps_ref[0, 0]) * xb
    t = jnp.dot(h, w1_ref[...], preferred_element_type=jnp.float32)
    t = _leaky(t + b1_ref[...])
    o = jnp.dot(t, w2_ref[...], preferred_element_type=jnp.float32)
    o = _leaky(o + b2_ref[...])
    grow = r * RB + lax.broadcasted_iota(jnp.int32, (RB, HID), 0)
    out_ref[0] = jnp.where(grow < N, o, 0.0)


def _conv_layer(a, x, w1, b1, w2, b2, eps):
    d = x.shape[-1]
    nr = NP // RB
    return pl.pallas_call(
        _conv_kernel,
        grid=(B, nr),
        in_specs=[
            pl.BlockSpec((1, RB, NP), lambda b, r: (b, r, 0)),
            pl.BlockSpec((1, NP, d), lambda b, r: (b, 0, 0)),
            pl.BlockSpec((d, HID), lambda b, r: (0, 0)),
            pl.BlockSpec((1, HID), lambda b, r: (0, 0)),
            pl.BlockSpec((HID, HID), lambda b, r: (0, 0)),
            pl.BlockSpec((1, HID), lambda b, r: (0, 0)),
            pl.BlockSpec((1, 128), lambda b, r: (0, 0)),
        ],
        out_specs=pl.BlockSpec((1, RB, HID), lambda b, r: (b, r, 0)),
        out_shape=jax.ShapeDtypeStruct((B, NP, HID), jnp.float32),
    )(a, x, w1, b1.reshape(1, HID), w2, b2.reshape(1, HID),
      jnp.full((1, 128), eps, jnp.float32))


def _final_kernel(x_ref, w1_ref, b1_ref, w2_ref, b2_ref, w3_ref, b3_ref,
                  out_ref):
    h = jnp.dot(x_ref[0], w1_ref[...], preferred_element_type=jnp.float32)
    h = _leaky(h + b1_ref[...])
    y = jnp.dot(h, w2_ref[...], preferred_element_type=jnp.float32)
    y = y + b2_ref[...]
    rows = lax.broadcasted_iota(jnp.int32, (NP, OUT_C), 0)
    y = jnp.where(rows < N, y, 0.0)
    g = jnp.sum(y, axis=0)
    val = jnp.sum(g * w3_ref[0]) + b3_ref[0, 0]
    out_ref[0] = jnp.full((8, 128), val, jnp.float32)


def _final_layers(x, lin1_W, lin1_b, lin2_W, lin2_b, lin3_W, lin3_b):
    out = pl.pallas_call(
        _final_kernel,
        grid=(B,),
        in_specs=[
            pl.BlockSpec((1, NP, HID), lambda b: (b, 0, 0)),
            pl.BlockSpec((HID, HID), lambda b: (0, 0)),
            pl.BlockSpec((1, HID), lambda b: (0, 0)),
            pl.BlockSpec((HID, OUT_C), lambda b: (0, 0)),
            pl.BlockSpec((1, OUT_C), lambda b: (0, 0)),
            pl.BlockSpec((1, OUT_C), lambda b: (0, 0)),
            pl.BlockSpec((1, 128), lambda b: (0, 0)),
        ],
        out_specs=pl.BlockSpec((1, 8, 128), lambda b: (b, 0, 0)),
        out_shape=jax.ShapeDtypeStruct((B, 8, 128), jnp.float32),
    )(x, lin1_W, lin1_b.reshape(1, HID), lin2_W, lin2_b.reshape(1, OUT_C),
      lin3_W.reshape(1, OUT_C),
      jnp.full((1, 128), lin3_b[0], jnp.float32))
    return out[:, 0, :1]


def kernel(graph_nodes, graph_edge_links, graph_edges,
           conv0_W1, conv0_b1, conv0_W2, conv0_b2, eps0,
           conv1_W1, conv1_b1, conv1_W2, conv1_b2, eps1,
           conv2_W1, conv2_b1, conv2_W2, conv2_b2, eps2,
           conv3_W1, conv3_b1, conv3_W2, conv3_b2, eps3,
           lin1_W, lin1_b, lin2_W, lin2_b, lin3_W, lin3_b):
    del graph_edges
    src = graph_edge_links[:, 0, :]
    dst = graph_edge_links[:, 1, :]
    flat = dst * NP + src
    flat = jnp.concatenate(
        [flat, jnp.full((B, EP - E), PAD_CELL, jnp.int32)], axis=1)
    flat = flat.reshape(B * NS, CHUNKS, 128)
    zeros_src = jnp.zeros((SLICE,), jnp.float32)

    a = _build_adjacency(flat, zeros_src)

    x = _conv_layer_first(a, graph_nodes, conv0_W1, conv0_b1,
                          conv0_W2, conv0_b2, eps0)
    x = _conv_layer(a, x, conv1_W1, conv1_b1, conv1_W2, conv1_b2, eps1)
    x = _conv_layer(a, x, conv2_W1, conv2_b1, conv2_W2, conv2_b2, eps2)
    x = _conv_layer(a, x, conv3_W1, conv3_b1, conv3_W2, conv3_b2, eps3)

    return _final_layers(x, lin1_W, lin1_b, lin2_W, lin2_b, lin3_W, lin3_b)


# speedup vs baseline: 1.2250x; 1.2250x over previous
import functools

import jax
import jax.numpy as jnp
from jax import lax
from jax.experimental import pallas as pl
from jax.experimental.pallas import tpu as pltpu
from jax.experimental.pallas import tpu_sc as plsc

B = 8
N = 1250
NP = 1280
E = 20000
EP = 20480
IN_C = 256
HID = 512
OUT_C = 256

NC = 2
NS = 16
NG = 4
GB = B // NG
CHUNKS = 10
CELLS = NP * NP
SLICE = CELLS // NS
PAD_CELL = CELLS


def _leaky(x):
    return jnp.where(x > 0, x, 0.01 * x)


def _build_adjacency(idx, zeros_src, grp):
    mesh = plsc.VectorSubcoreMesh(
        core_axis_name="c", subcore_axis_name="s",
        num_cores=NC, num_subcores=NS)

    @functools.partial(
        pl.kernel,
        mesh=mesh,
        out_type=jax.ShapeDtypeStruct((GB, NP, NP), jnp.float32),
        scratch_types=[
            pltpu.VMEM((CHUNKS, 128), jnp.int32),
            pltpu.VMEM((128,), jnp.float32),
            pltpu.VMEM((16, NP), jnp.float32),
            pltpu.VMEM_SHARED((CELLS + 16,), jnp.float32),
        ],
    )
    def build(idx_hbm, zeros_hbm, a_hbm, idx_v, ones_v, row_v, a_sh):
        cid = lax.axis_index("c")
        sid = lax.axis_index("s")
        rows = NP // NS
        for i in range(8):
            ones_v[pl.ds(16 * i, 16)] = jnp.ones((16,), jnp.float32)
        pltpu.sync_copy(zeros_hbm, a_sh.at[pl.ds(sid * SLICE, SLICE)])
        plsc.subcore_barrier()
        b = grp * GB + cid
        pltpu.sync_copy(idx_hbm.at[b * NS + sid], idx_v)
        for j in range(CHUNKS):
            pltpu.sync_copy(ones_v, a_sh.at[idx_v.at[j]], add=True)
        plsc.subcore_barrier()
        for g in range(rows // 16):
            def _row(r, _):
                pltpu.sync_copy(
                    a_sh.at[pl.ds((sid * rows + g * 16 + r) * NP, NP)],
                    row_v.at[r])
                return _
            lax.fori_loop(0, 16, _row, 0)
            pltpu.sync_copy(
                row_v, a_hbm.at[cid].at[pl.ds(sid * rows + g * 16, 16)])

    return build(idx, zeros_src)


RB = 640


def _conv_kernel_first(a_ref, xf_ref, w1_ref, b1_ref, w2_ref, b2_ref,
                       eps_ref, out_ref):
    xf = xf_ref[0]
    aggr = jnp.dot(a_ref[0][:N, :N], xf, preferred_element_type=jnp.float32)
    h = aggr + (1.0 + eps_ref[0, 0]) * xf
    t = jnp.dot(h, w1_ref[...], preferred_element_type=jnp.float32)
    t = _leaky(t + b1_ref[...])
    o = jnp.dot(t, w2_ref[...], preferred_element_type=jnp.float32)
    o = _leaky(o + b2_ref[...])
    out_ref[0] = jnp.concatenate(
        [o, jnp.zeros((NP - N, HID), jnp.float32)], axis=0)


def _conv_layer_first(a, x, w1, b1, w2, b2, eps, grp):
    d = x.shape[-1]
    g0 = grp * GB
    return pl.pallas_call(
        _conv_kernel_first,
        grid=(GB,),
        in_specs=[
            pl.BlockSpec((1, NP, NP), lambda b: (b, 0, 0)),
            pl.BlockSpec((1, N, d), lambda b, g0=g0: (g0 + b, 0, 0)),
            pl.BlockSpec((d, HID), lambda b: (0, 0)),
            pl.BlockSpec((1, HID), lambda b: (0, 0)),
            pl.BlockSpec((HID, HID), lambda b: (0, 0)),
            pl.BlockSpec((1, HID), lambda b: (0, 0)),
            pl.BlockSpec((1, 128), lambda b: (0, 0)),
        ],
        out_specs=pl.BlockSpec((1, NP, HID), lambda b: (b, 0, 0)),
        out_shape=jax.ShapeDtypeStruct((GB, NP, HID), jnp.float32),
    )(a, x, w1, b1.reshape(1, HID), w2, b2.reshape(1, HID),
      jnp.full((1, 128), eps, jnp.float32))


def _conv_kernel(a_ref, xf_ref, w1_ref, b1_ref, w2_ref, b2_ref,
                 eps_ref, out_ref):
    r = pl.program_id(1)
    xf = xf_ref[0]
    aggr = jnp.dot(a_ref[0], xf, preferred_element_type=jnp.float32)
    xb = xf_ref[0, pl.ds(r * RB, RB), :]
    h = aggr + (1.0 + eps_ref[0, 0]) * xb
    t = jnp.dot(h, w1_ref[...], preferred_element_type=jnp.float32)
    t = _leaky(t + b1_ref[...])
    o = jnp.dot(t, w2_ref[...], preferred_element_type=jnp.float32)
    o = _leaky(o + b2_ref[...])
    grow = r * RB + lax.broadcasted_iota(jnp.int32, (RB, HID), 0)
    out_ref[0] = jnp.where(grow < N, o, 0.0)


def _conv_layer(a, x, w1, b1, w2, b2, eps):
    d = x.shape[-1]
    nr = NP // RB
    return pl.pallas_call(
        _conv_kernel,
        grid=(GB, nr),
        in_specs=[
            pl.BlockSpec((1, RB, NP), lambda b, r: (b, r, 0)),
            pl.BlockSpec((1, NP, d), lambda b, r: (b, 0, 0)),
            pl.BlockSpec((d, HID), lambda b, r: (0, 0)),
            pl.BlockSpec((1, HID), lambda b, r: (0, 0)),
            pl.BlockSpec((HID, HID), lambda b, r: (0, 0)),
            pl.BlockSpec((1, HID), lambda b, r: (0, 0)),
            pl.BlockSpec((1, 128), lambda b, r: (0, 0)),
        ],
        out_specs=pl.BlockSpec((1, RB, HID), lambda b, r: (b, r, 0)),
        out_shape=jax.ShapeDtypeStruct((GB, NP, HID), jnp.float32),
    )(a, x, w1, b1.reshape(1, HID), w2, b2.reshape(1, HID),
      jnp.full((1, 128), eps, jnp.float32))


def _final_kernel(x_ref, w1_ref, b1_ref, w2_ref, b2_ref, w3_ref, b3_ref,
                  out_ref):
    h = jnp.dot(x_ref[0], w1_ref[...], preferred_element_type=jnp.float32)
    h = _leaky(h + b1_ref[...])
    y = jnp.dot(h, w2_ref[...], preferred_element_type=jnp.float32)
    y = y + b2_ref[...]
    rows = lax.broadcasted_iota(jnp.int32, (NP, OUT_C), 0)
    y = jnp.where(rows < N, y, 0.0)
    g = jnp.sum(y, axis=0)
    val = jnp.sum(g * w3_ref[0]) + b3_ref[0, 0]
    out_ref[0] = jnp.full((8, 128), val, jnp.float32)


def _final_layers(x, lin1_W, lin1_b, lin2_W, lin2_b, lin3_W, lin3_b):
    out = pl.pallas_call(
        _final_kernel,
        grid=(GB,),
        in_specs=[
            pl.BlockSpec((1, NP, HID), lambda b: (b, 0, 0)),
            pl.BlockSpec((HID, HID), lambda b: (0, 0)),
            pl.BlockSpec((1, HID), lambda b: (0, 0)),
            pl.BlockSpec((HID, OUT_C), lambda b: (0, 0)),
            pl.BlockSpec((1, OUT_C), lambda b: (0, 0)),
            pl.BlockSpec((1, OUT_C), lambda b: (0, 0)),
            pl.BlockSpec((1, 128), lambda b: (0, 0)),
        ],
        out_specs=pl.BlockSpec((1, 8, 128), lambda b: (b, 0, 0)),
        out_shape=jax.ShapeDtypeStruct((GB, 8, 128), jnp.float32),
    )(x, lin1_W, lin1_b.reshape(1, HID), lin2_W, lin2_b.reshape(1, OUT_C),
      lin3_W.reshape(1, OUT_C),
      jnp.full((1, 128), lin3_b[0], jnp.float32))
    return out[:, 0, :1]


def kernel(graph_nodes, graph_edge_links, graph_edges,
           conv0_W1, conv0_b1, conv0_W2, conv0_b2, eps0,
           conv1_W1, conv1_b1, conv1_W2, conv1_b2, eps1,
           conv2_W1, conv2_b1, conv2_W2, conv2_b2, eps2,
           conv3_W1, conv3_b1, conv3_W2, conv3_b2, eps3,
           lin1_W, lin1_b, lin2_W, lin2_b, lin3_W, lin3_b):
    del graph_edges
    src = graph_edge_links[:, 0, :]
    dst = graph_edge_links[:, 1, :]
    flat = dst * NP + src
    flat = jnp.concatenate(
        [flat, jnp.full((B, EP - E), PAD_CELL, jnp.int32)], axis=1)
    flat = flat.reshape(B * NS, CHUNKS, 128)
    zeros_src = jnp.zeros((SLICE,), jnp.float32)

    a_gs = [_build_adjacency(flat, zeros_src, g) for g in range(NG)]

    outs = []
    for g in range(NG):
        x = _conv_layer_first(a_gs[g], graph_nodes, conv0_W1, conv0_b1,
                              conv0_W2, conv0_b2, eps0, g)
        x = _conv_layer(a_gs[g], x, conv1_W1, conv1_b1, conv1_W2, conv1_b2,
                        eps1)
        x = _conv_layer(a_gs[g], x, conv2_W1, conv2_b1, conv2_W2, conv2_b2,
                        eps2)
        x = _conv_layer(a_gs[g], x, conv3_W1, conv3_b1, conv3_W2, conv3_b2,
                        eps3)
        outs.append(_final_layers(x, lin1_W, lin1_b, lin2_W, lin2_b,
                                  lin3_W, lin3_b))
    return jnp.concatenate(outs, axis=0)


# speedup vs baseline: 1.2855x; 1.0494x over previous
import functools

import jax
import jax.numpy as jnp
from jax import lax
from jax.experimental import pallas as pl
from jax.experimental.pallas import tpu as pltpu
from jax.experimental.pallas import tpu_sc as plsc

B = 8
N = 1250
NP = 1280
E = 20000
EP = 20480
IN_C = 256
HID = 512
OUT_C = 256

NC = 2
NS = 16
NG = 2
GB = B // NG
CHUNKS = 10
CELLS = NP * NP
SLICE = CELLS // NS
PAD_CELL = CELLS


def _leaky(x):
    return jnp.where(x > 0, x, 0.01 * x)


def _build_adjacency(idx, zeros_src, grp):
    mesh = plsc.VectorSubcoreMesh(
        core_axis_name="c", subcore_axis_name="s",
        num_cores=NC, num_subcores=NS)

    @functools.partial(
        pl.kernel,
        mesh=mesh,
        out_type=jax.ShapeDtypeStruct((GB, NP, NP), jnp.float32),
        scratch_types=[
            pltpu.VMEM((CHUNKS, 128), jnp.int32),
            pltpu.VMEM((128,), jnp.float32),
            pltpu.VMEM((16, NP), jnp.float32),
            pltpu.VMEM_SHARED((CELLS + 16,), jnp.float32),
        ],
    )
    def build(idx_hbm, zeros_hbm, a_hbm, idx_v, ones_v, row_v, a_sh):
        cid = lax.axis_index("c")
        sid = lax.axis_index("s")
        rows = NP // NS
        for i in range(8):
            ones_v[pl.ds(16 * i, 16)] = jnp.ones((16,), jnp.float32)
        pltpu.sync_copy(zeros_hbm, a_sh.at[pl.ds(sid * SLICE, SLICE)])
        plsc.subcore_barrier()
        for bi in range(GB // NC):
            b = grp * GB + cid * (GB // NC) + bi
            bo = cid * (GB // NC) + bi
            pltpu.sync_copy(idx_hbm.at[b * NS + sid], idx_v)
            for j in range(CHUNKS):
                pltpu.sync_copy(ones_v, a_sh.at[idx_v.at[j]], add=True)
            plsc.subcore_barrier()
            for g in range(rows // 16):
                def _row(r, _):
                    pltpu.sync_copy(
                        a_sh.at[pl.ds((sid * rows + g * 16 + r) * NP, NP)],
                        row_v.at[r])
                    return _
                lax.fori_loop(0, 16, _row, 0)
                pltpu.sync_copy(
                    row_v, a_hbm.at[bo].at[pl.ds(sid * rows + g * 16, 16)])
            if bi != GB // NC - 1:
                pltpu.sync_copy(zeros_hbm, a_sh.at[pl.ds(sid * SLICE, SLICE)])
                plsc.subcore_barrier()

    return build(idx, zeros_src)


RB = 640


def _conv_kernel_first(a_ref, xf_ref, w1_ref, b1_ref, w2_ref, b2_ref,
                       eps_ref, out_ref):
    xf = xf_ref[0]
    aggr = jnp.dot(a_ref[0][:N, :N], xf, preferred_element_type=jnp.float32)
    h = aggr + (1.0 + eps_ref[0, 0]) * xf
    t = jnp.dot(h, w1_ref[...], preferred_element_type=jnp.float32)
    t = _leaky(t + b1_ref[...])
    o = jnp.dot(t, w2_ref[...], preferred_element_type=jnp.float32)
    o = _leaky(o + b2_ref[...])
    out_ref[0] = jnp.concatenate(
        [o, jnp.zeros((NP - N, HID), jnp.float32)], axis=0)


def _conv_layer_first(a, x, w1, b1, w2, b2, eps, grp):
    d = x.shape[-1]
    g0 = grp * GB
    return pl.pallas_call(
        _conv_kernel_first,
        grid=(GB,),
        in_specs=[
            pl.BlockSpec((1, NP, NP), lambda b: (b, 0, 0)),
            pl.BlockSpec((1, N, d), lambda b, g0=g0: (g0 + b, 0, 0)),
            pl.BlockSpec((d, HID), lambda b: (0, 0)),
            pl.BlockSpec((1, HID), lambda b: (0, 0)),
            pl.BlockSpec((HID, HID), lambda b: (0, 0)),
            pl.BlockSpec((1, HID), lambda b: (0, 0)),
            pl.BlockSpec((1, 128), lambda b: (0, 0)),
        ],
        out_specs=pl.BlockSpec((1, NP, HID), lambda b: (b, 0, 0)),
        out_shape=jax.ShapeDtypeStruct((GB, NP, HID), jnp.float32),
    )(a, x, w1, b1.reshape(1, HID), w2, b2.reshape(1, HID),
      jnp.full((1, 128), eps, jnp.float32))


def _conv_kernel(a_ref, xf_ref, w1_ref, b1_ref, w2_ref, b2_ref,
                 eps_ref, out_ref):
    r = pl.program_id(1)
    xf = xf_ref[0]
    aggr = jnp.dot(a_ref[0], xf, preferred_element_type=jnp.float32)
    xb = xf_ref[0, pl.ds(r * RB, RB), :]
    h = aggr + (1.0 + eps_ref[0, 0]) * xb
    t = jnp.dot(h, w1_ref[...], preferred_element_type=jnp.float32)
    t = _leaky(t + b1_ref[...])
    o = jnp.dot(t, w2_ref[...], preferred_element_type=jnp.float32)
    o = _leaky(o + b2_ref[...])
    grow = r * RB + lax.broadcasted_iota(jnp.int32, (RB, HID), 0)
    out_ref[0] = jnp.where(grow < N, o, 0.0)


def _conv_layer(a, x, w1, b1, w2, b2, eps):
    d = x.shape[-1]
    nr = NP // RB
    return pl.pallas_call(
        _conv_kernel,
        grid=(GB, nr),
        in_specs=[
            pl.BlockSpec((1, RB, NP), lambda b, r: (b, r, 0)),
            pl.BlockSpec((1, NP, d), lambda b, r: (b, 0, 0)),
            pl.BlockSpec((d, HID), lambda b, r: (0, 0)),
            pl.BlockSpec((1, HID), lambda b, r: (0, 0)),
            pl.BlockSpec((HID, HID), lambda b, r: (0, 0)),
            pl.BlockSpec((1, HID), lambda b, r: (0, 0)),
            pl.BlockSpec((1, 128), lambda b, r: (0, 0)),
        ],
        out_specs=pl.BlockSpec((1, RB, HID), lambda b, r: (b, r, 0)),
        out_shape=jax.ShapeDtypeStruct((GB, NP, HID), jnp.float32),
    )(a, x, w1, b1.reshape(1, HID), w2, b2.reshape(1, HID),
      jnp.full((1, 128), eps, jnp.float32))


def _final_kernel(x_ref, w1_ref, b1_ref, w2_ref, b2_ref, w3_ref, b3_ref,
                  out_ref):
    h = jnp.dot(x_ref[0], w1_ref[...], preferred_element_type=jnp.float32)
    h = _leaky(h + b1_ref[...])
    y = jnp.dot(h, w2_ref[...], preferred_element_type=jnp.float32)
    y = y + b2_ref[...]
    rows = lax.broadcasted_iota(jnp.int32, (NP, OUT_C), 0)
    y = jnp.where(rows < N, y, 0.0)
    g = jnp.sum(y, axis=0)
    val = jnp.sum(g * w3_ref[0]) + b3_ref[0, 0]
    out_ref[0] = jnp.full((8, 128), val, jnp.float32)


def _final_layers(x, lin1_W, lin1_b, lin2_W, lin2_b, lin3_W, lin3_b):
    out = pl.pallas_call(
        _final_kernel,
        grid=(GB,),
        in_specs=[
            pl.BlockSpec((1, NP, HID), lambda b: (b, 0, 0)),
            pl.BlockSpec((HID, HID), lambda b: (0, 0)),
            pl.BlockSpec((1, HID), lambda b: (0, 0)),
            pl.BlockSpec((HID, OUT_C), lambda b: (0, 0)),
            pl.BlockSpec((1, OUT_C), lambda b: (0, 0)),
            pl.BlockSpec((1, OUT_C), lambda b: (0, 0)),
            pl.BlockSpec((1, 128), lambda b: (0, 0)),
        ],
        out_specs=pl.BlockSpec((1, 8, 128), lambda b: (b, 0, 0)),
        out_shape=jax.ShapeDtypeStruct((GB, 8, 128), jnp.float32),
    )(x, lin1_W, lin1_b.reshape(1, HID), lin2_W, lin2_b.reshape(1, OUT_C),
      lin3_W.reshape(1, OUT_C),
      jnp.full((1, 128), lin3_b[0], jnp.float32))
    return out[:, 0, :1]


def kernel(graph_nodes, graph_edge_links, graph_edges,
           conv0_W1, conv0_b1, conv0_W2, conv0_b2, eps0,
           conv1_W1, conv1_b1, conv1_W2, conv1_b2, eps1,
           conv2_W1, conv2_b1, conv2_W2, conv2_b2, eps2,
           conv3_W1, conv3_b1, conv3_W2, conv3_b2, eps3,
           lin1_W, lin1_b, lin2_W, lin2_b, lin3_W, lin3_b):
    del graph_edges
    src = graph_edge_links[:, 0, :]
    dst = graph_edge_links[:, 1, :]
    flat = dst * NP + src
    flat = jnp.concatenate(
        [flat, jnp.full((B, EP - E), PAD_CELL, jnp.int32)], axis=1)
    flat = flat.reshape(B * NS, CHUNKS, 128)
    zeros_src = jnp.zeros((SLICE,), jnp.float32)

    a_gs = [_build_adjacency(flat, zeros_src, g) for g in range(NG)]

    outs = []
    for g in range(NG):
        x = _conv_layer_first(a_gs[g], graph_nodes, conv0_W1, conv0_b1,
                              conv0_W2, conv0_b2, eps0, g)
        x = _conv_layer(a_gs[g], x, conv1_W1, conv1_b1, conv1_W2, conv1_b2,
                        eps1)
        x = _conv_layer(a_gs[g], x, conv2_W1, conv2_b1, conv2_W2, conv2_b2,
                        eps2)
        x = _conv_layer(a_gs[g], x, conv3_W1, conv3_b1, conv3_W2, conv3_b2,
                        eps3)
        outs.append(_final_layers(x, lin1_W, lin1_b, lin2_W, lin2_b,
                                  lin3_W, lin3_b))
    return jnp.concatenate(outs, axis=0)
